# baseline (device time: 7351 ns/iter reference)
import jax
import jax.numpy as jnp
from jax import lax
from jax.experimental import pallas as pl
from jax.experimental.pallas import tpu as pltpu

N_DEV = 4


def kernel(x):
    m_rows, n_cols = x.shape

    def body(x_hbm, out_hbm, xv_ref, ov_ref, comm_ref,
             load_sem, store_sem, send_sems, recv_sems):
        my = lax.axis_index("i")

        load = pltpu.make_async_copy(x_hbm, xv_ref, load_sem)
        load.start()

        barrier_sem = pltpu.get_barrier_semaphore()
        for k in range(1, N_DEV):
            pl.semaphore_signal(
                barrier_sem, inc=1,
                device_id=((my + k) % N_DEV,),
                device_id_type=pl.DeviceIdType.MESH,
            )
        load.wait()

        xv = xv_ref[:, :]
        m = jnp.max(xv, axis=1, keepdims=True)
        s = jnp.sum(jnp.exp(xv - m), axis=1, keepdims=True)
        comm_ref[0, :, :] = jnp.transpose(
            jnp.concatenate([m, s], axis=1)
        )

        pl.semaphore_wait(barrier_sem, N_DEV - 1)

        rdmas = []
        for k in range(1, N_DEV):
            rdma = pltpu.make_async_remote_copy(
                src_ref=comm_ref.at[0],
                dst_ref=comm_ref.at[k],
                send_sem=send_sems.at[k - 1],
                recv_sem=recv_sems.at[k - 1],
                device_id=((my + k) % N_DEV,),
                device_id_type=pl.DeviceIdType.MESH,
            )
            rdma.start()
            rdmas.append(rdma)
        for rdma in rdmas:
            rdma.wait_recv()

        stats = comm_ref[:, :, :]
        m_all = stats[:, 0, :]
        s_all = stats[:, 1, :]
        gmax = jnp.max(m_all, axis=0, keepdims=True)
        gsum = jnp.sum(s_all * jnp.exp(m_all - gmax), axis=0,
                       keepdims=True)
        c = jnp.transpose(gmax + jnp.log(gsum))

        half = m_rows // 2
        ov_ref[0:half, :] = jnp.exp(xv[0:half, :] - c[0:half, :])
        st0 = pltpu.make_async_copy(
            ov_ref.at[0:half], out_hbm.at[0:half], store_sem.at[0]
        )
        st0.start()
        ov_ref[half:, :] = jnp.exp(xv[half:, :] - c[half:, :])
        st1 = pltpu.make_async_copy(
            ov_ref.at[half:], out_hbm.at[half:], store_sem.at[1]
        )
        st1.start()
        st0.wait()
        st1.wait()

        for rdma in rdmas:
            rdma.wait_send()

    return pl.pallas_call(
        body,
        out_shape=jax.ShapeDtypeStruct((m_rows, n_cols), jnp.float32),
        in_specs=[pl.BlockSpec(memory_space=pl.ANY)],
        out_specs=pl.BlockSpec(memory_space=pl.ANY),
        scratch_shapes=[
            pltpu.VMEM((m_rows, n_cols), jnp.float32),
            pltpu.VMEM((m_rows, n_cols), jnp.float32),
            pltpu.VMEM((N_DEV, 2, m_rows), jnp.float32),
            pltpu.SemaphoreType.DMA,
            pltpu.SemaphoreType.DMA((2,)),
            pltpu.SemaphoreType.DMA((N_DEV - 1,)),
            pltpu.SemaphoreType.DMA((N_DEV - 1,)),
        ],
        compiler_params=pltpu.CompilerParams(collective_id=0),
    )(x)


# device time: 7343 ns/iter; 1.0011x vs baseline; 1.0011x over previous
import jax
import jax.numpy as jnp
from jax import lax
from jax.experimental import pallas as pl
from jax.experimental.pallas import tpu as pltpu

N_DEV = 4


def kernel(x):
    m_rows, n_cols = x.shape

    def body(x_hbm, out_hbm, xv_ref, ov_ref, comm_ref,
             load_sem, store_sem, send_sems, recv_sems):
        my = lax.axis_index("i")

        load = pltpu.make_async_copy(x_hbm, xv_ref, load_sem)
        load.start()

        barrier_sem = pltpu.get_barrier_semaphore()
        for k in range(1, N_DEV):
            pl.semaphore_signal(
                barrier_sem, inc=1,
                device_id=((my + k) % N_DEV,),
                device_id_type=pl.DeviceIdType.MESH,
            )
        load.wait()

        xv = xv_ref[:, :]
        m = jnp.max(xv, axis=1, keepdims=True)
        s = jnp.sum(jnp.exp(xv - m), axis=1, keepdims=True)
        comm_ref[0, :, :] = jnp.transpose(
            jnp.concatenate([m, s], axis=1)
        )

        pl.semaphore_wait(barrier_sem, N_DEV - 1)

        rdmas = []
        for k in range(1, N_DEV):
            rdma = pltpu.make_async_remote_copy(
                src_ref=comm_ref.at[0],
                dst_ref=comm_ref.at[k],
                send_sem=send_sems.at[k - 1],
                recv_sem=recv_sems.at[k - 1],
                device_id=((my + k) % N_DEV,),
                device_id_type=pl.DeviceIdType.MESH,
            )
            rdma.start()
            rdmas.append(rdma)
        for rdma in rdmas:
            rdma.wait_recv()

        stats = comm_ref[:, :, :]
        m_all = stats[:, 0, :]
        s_all = stats[:, 1, :]
        gmax = jnp.max(m_all, axis=0, keepdims=True)
        gsum = jnp.sum(s_all * jnp.exp(m_all - gmax), axis=0,
                       keepdims=True)
        c = jnp.transpose(gmax + jnp.log(gsum))

        half = m_rows // 2
        ov_ref[0:half, :] = jnp.exp(xv[0:half, :] - c[0:half, :])
        st0 = pltpu.make_async_copy(
            ov_ref.at[0:half], out_hbm.at[0:half], store_sem.at[0]
        )
        st0.start()
        ov_ref[half:, :] = jnp.exp(xv[half:, :] - c[half:, :])
        st1 = pltpu.make_async_copy(
            ov_ref.at[half:], out_hbm.at[half:], store_sem.at[1]
        )
        st1.start()
        st0.wait()
        st1.wait()

        for rdma in rdmas:
            rdma.wait_send()

    return pl.pallas_call(
        body,
        out_shape=jax.ShapeDtypeStruct((m_rows, n_cols), jnp.float32),
        in_specs=[pl.BlockSpec(memory_space=pltpu.MemorySpace.HBM)],
        out_specs=pl.BlockSpec(memory_space=pltpu.MemorySpace.HBM),
        scratch_shapes=[
            pltpu.VMEM((m_rows, n_cols), jnp.float32),
            pltpu.VMEM((m_rows, n_cols), jnp.float32),
            pltpu.VMEM((N_DEV, 2, m_rows), jnp.float32),
            pltpu.SemaphoreType.DMA,
            pltpu.SemaphoreType.DMA((2,)),
            pltpu.SemaphoreType.DMA((N_DEV - 1,)),
            pltpu.SemaphoreType.DMA((N_DEV - 1,)),
        ],
        compiler_params=pltpu.CompilerParams(collective_id=0),
    )(x)


# device time: 7150 ns/iter; 1.0281x vs baseline; 1.0270x over previous
import jax
import jax.numpy as jnp
from jax import lax
from jax.experimental import pallas as pl
from jax.experimental.pallas import tpu as pltpu

N_DEV = 4


def kernel(x):
    m_rows, n_cols = x.shape

    def body(x_ref, out_ref, comm_ref, send_sems, recv_sems):
        my = lax.axis_index("i")

        barrier_sem = pltpu.get_barrier_semaphore()
        for k in range(1, N_DEV):
            pl.semaphore_signal(
                barrier_sem, inc=1,
                device_id=((my + k) % N_DEV,),
                device_id_type=pl.DeviceIdType.MESH,
            )

        xv = x_ref[:, :]
        m = jnp.max(xv, axis=1, keepdims=True)
        s = jnp.sum(jnp.exp(xv - m), axis=1, keepdims=True)
        comm_ref[0, :, :] = jnp.transpose(
            jnp.concatenate([m, s], axis=1)
        )

        pl.semaphore_wait(barrier_sem, N_DEV - 1)

        rdmas = []
        for k in range(1, N_DEV):
            rdma = pltpu.make_async_remote_copy(
                src_ref=comm_ref.at[0],
                dst_ref=comm_ref.at[k],
                send_sem=send_sems.at[k - 1],
                recv_sem=recv_sems.at[k - 1],
                device_id=((my + k) % N_DEV,),
                device_id_type=pl.DeviceIdType.MESH,
            )
            rdma.start()
            rdmas.append(rdma)
        for rdma in rdmas:
            rdma.wait_recv()

        stats = comm_ref[:, :, :]
        m_all = stats[:, 0, :]
        s_all = stats[:, 1, :]
        gmax = jnp.max(m_all, axis=0, keepdims=True)
        gsum = jnp.sum(s_all * jnp.exp(m_all - gmax), axis=0,
                       keepdims=True)
        c = jnp.transpose(gmax + jnp.log(gsum))

        out_ref[:, :] = jnp.exp(xv - c)

        for rdma in rdmas:
            rdma.wait_send()

    return pl.pallas_call(
        body,
        out_shape=jax.ShapeDtypeStruct((m_rows, n_cols), jnp.float32),
        in_specs=[pl.BlockSpec(memory_space=pltpu.VMEM)],
        out_specs=pl.BlockSpec(memory_space=pltpu.VMEM),
        scratch_shapes=[
            pltpu.VMEM((N_DEV, 2, m_rows), jnp.float32),
            pltpu.SemaphoreType.DMA((N_DEV - 1,)),
            pltpu.SemaphoreType.DMA((N_DEV - 1,)),
        ],
        compiler_params=pltpu.CompilerParams(collective_id=0),
    )(x)


# device time: 7142 ns/iter; 1.0293x vs baseline; 1.0011x over previous
import jax
import jax.numpy as jnp
from jax import lax
from jax.experimental import pallas as pl
from jax.experimental.pallas import tpu as pltpu

N_DEV = 4


def kernel(x):
    m_rows, n_cols = x.shape

    def body(x_ref, out_ref, comm_ref, send_sems, recv_sems):
        my = lax.axis_index("i")

        barrier_sem = pltpu.get_barrier_semaphore()
        for k in range(1, N_DEV):
            pl.semaphore_signal(
                barrier_sem, inc=1,
                device_id=((my + k) % N_DEV,),
                device_id_type=pl.DeviceIdType.MESH,
            )

        xv = x_ref[:, :]
        m = jnp.max(xv, axis=1, keepdims=True)
        s = jnp.sum(jnp.exp(xv - m), axis=1, keepdims=True)
        comm_ref[0, :, :] = jnp.transpose(
            jnp.concatenate([m, s], axis=1)
        )

        pl.semaphore_wait(barrier_sem, N_DEV - 1)

        rdmas = []
        for k in range(1, N_DEV):
            rdma = pltpu.make_async_remote_copy(
                src_ref=comm_ref.at[0],
                dst_ref=comm_ref.at[k],
                send_sem=send_sems.at[k - 1],
                recv_sem=recv_sems.at[k - 1],
                device_id=((my + k) % N_DEV,),
                device_id_type=pl.DeviceIdType.MESH,
            )
            rdma.start()
            rdmas.append(rdma)
        rdmas[0].wait_recv()
        rdmas[2].wait_recv()
        near = comm_ref[:, :, :]
        m3 = jnp.max(near[0:2, 0, :], axis=0,
                     keepdims=True)
        m3 = jnp.maximum(m3, near[3:4, 0, :])
        e3 = (near[0:2, 1, :] * jnp.exp(near[0:2, 0, :] - m3))
        s3 = jnp.sum(e3, axis=0, keepdims=True) \
            + near[3, 1, :] * jnp.exp(near[3:4, 0, :] - m3)

        rdmas[1].wait_recv()
        md = comm_ref[2, 0:1, :]
        sd = comm_ref[2, 1:2, :]
        gmax = jnp.maximum(m3, md)
        gsum = s3 * jnp.exp(m3 - gmax) + sd * jnp.exp(md - gmax)
        c = jnp.transpose(gmax + jnp.log(gsum))

        out_ref[:, :] = jnp.exp(xv - c)

        for rdma in rdmas:
            rdma.wait_send()

    return pl.pallas_call(
        body,
        out_shape=jax.ShapeDtypeStruct((m_rows, n_cols), jnp.float32),
        in_specs=[pl.BlockSpec(memory_space=pltpu.VMEM)],
        out_specs=pl.BlockSpec(memory_space=pltpu.VMEM),
        scratch_shapes=[
            pltpu.VMEM((N_DEV, 2, m_rows), jnp.float32),
            pltpu.SemaphoreType.DMA((N_DEV - 1,)),
            pltpu.SemaphoreType.DMA((N_DEV - 1,)),
        ],
        compiler_params=pltpu.CompilerParams(collective_id=0),
    )(x)
